# baseline (speedup 1.0000x reference)
import functools

import jax
import jax.numpy as jnp
from jax import lax
from jax.experimental import pallas as pl
from jax.experimental.pallas import tpu as pltpu


def _qhead_kernel(x_ref, w_ref, b_ref, o_ref, *, repack, n_out, k_in):
    rn, rk = repack * n_out, repack * k_in
    w_tiled = jnp.tile(w_ref[...], (repack, repack))
    row_grp = lax.broadcasted_iota(jnp.int32, (rn, rk), 0) // n_out
    col_grp = lax.broadcasted_iota(jnp.int32, (rn, rk), 1) // k_in
    w_bd = jnp.where(row_grp == col_grp, w_tiled, jnp.float32(0.0))
    acc = lax.dot_general(
        x_ref[...], w_bd,
        dimension_numbers=(((1,), (1,)), ((), ())),
        preferred_element_type=jnp.float32,
    )
    o_ref[...] = (acc + jnp.tile(b_ref[...], (1, repack))).astype(o_ref.dtype)


def kernel(x, weight, bias):
    B, K = x.shape
    N = weight.shape[0]
    R = 128 // N
    Be, Ke, Ne = B // R, R * K, R * N

    x_eff = x.reshape(Be, Ke)

    tm = 2048
    while Be % tm:
        tm //= 2
    grid = (Be // tm,)

    body = functools.partial(_qhead_kernel, repack=R, n_out=N, k_in=K)
    out_eff = pl.pallas_call(
        body,
        out_shape=jax.ShapeDtypeStruct((Be, Ne), x.dtype),
        grid=grid,
        in_specs=[
            pl.BlockSpec((tm, Ke), lambda i: (i, 0)),
            pl.BlockSpec((N, K), lambda i: (0, 0)),
            pl.BlockSpec((1, N), lambda i: (0, 0)),
        ],
        out_specs=pl.BlockSpec((tm, Ne), lambda i: (i, 0)),
        compiler_params=pltpu.CompilerParams(
            dimension_semantics=("parallel",),
            vmem_limit_bytes=64 * 1024 * 1024,
        ),
    )(x_eff, weight, bias.reshape(1, N))

    return out_eff.reshape(B, N)


# speedup vs baseline: 1.2193x; 1.2193x over previous
import jax
import jax.numpy as jnp
from jax import lax
from jax.experimental import pallas as pl
from jax.experimental.pallas import tpu as pltpu


def _qhead_kernel(x_ref, w_ref, b_ref, o_ref):
    acc = lax.dot_general(
        x_ref[...], w_ref[...],
        dimension_numbers=(((1,), (1,)), ((), ())),
        preferred_element_type=jnp.float32,
    )
    o_ref[...] = (acc + b_ref[...]).astype(o_ref.dtype)


def kernel(x, weight, bias):
    B, K = x.shape
    N = weight.shape[0]

    tm = 8192
    while B % tm:
        tm //= 2
    grid = (B // tm,)

    out = pl.pallas_call(
        _qhead_kernel,
        out_shape=jax.ShapeDtypeStruct((B, N), x.dtype),
        grid=grid,
        in_specs=[
            pl.BlockSpec((tm, K), lambda i: (i, 0)),
            pl.BlockSpec((N, K), lambda i: (0, 0)),
            pl.BlockSpec((1, N), lambda i: (0, 0)),
        ],
        out_specs=pl.BlockSpec((tm, N), lambda i: (i, 0)),
        compiler_params=pltpu.CompilerParams(
            dimension_semantics=("parallel",),
            vmem_limit_bytes=64 * 1024 * 1024,
        ),
    )(x, weight, bias.reshape(1, N))

    return out


# speedup vs baseline: 17.0207x; 13.9597x over previous
import jax
import jax.numpy as jnp
from jax import lax
from jax.experimental import pallas as pl
from jax.experimental.pallas import tpu as pltpu


def _qhead_kernel(x_ref, w_ref, b_ref, o_ref):
    acc = lax.dot_general(
        w_ref[...], x_ref[...],
        dimension_numbers=(((1,), (0,)), ((), ())),
        preferred_element_type=jnp.float32,
    )
    o_ref[...] = (acc + b_ref[...]).astype(o_ref.dtype)


def kernel(x, weight, bias):
    B, K = x.shape
    N = weight.shape[0]

    xt = x.T

    tn = 32768
    while B % tn:
        tn //= 2
    grid = (B // tn,)

    outt = pl.pallas_call(
        _qhead_kernel,
        out_shape=jax.ShapeDtypeStruct((N, B), x.dtype),
        grid=grid,
        in_specs=[
            pl.BlockSpec((K, tn), lambda i: (0, i)),
            pl.BlockSpec((N, K), lambda i: (0, 0)),
            pl.BlockSpec((N, 1), lambda i: (0, 0)),
        ],
        out_specs=pl.BlockSpec((N, tn), lambda i: (0, i)),
        compiler_params=pltpu.CompilerParams(
            dimension_semantics=("parallel",),
            vmem_limit_bytes=64 * 1024 * 1024,
        ),
    )(xt, weight, bias.reshape(N, 1))

    return outt.T


# speedup vs baseline: 18.2547x; 1.0725x over previous
import jax
import jax.numpy as jnp
from jax import lax
from jax.experimental import pallas as pl
from jax.experimental.pallas import tpu as pltpu


def _qhead_kernel(x_ref, w_ref, b_ref, o_ref):
    acc = lax.dot_general(
        w_ref[...], x_ref[...],
        dimension_numbers=(((1,), (0,)), ((), ())),
        preferred_element_type=jnp.float32,
    )
    b_col = jnp.transpose(b_ref[...])
    o_ref[...] = (acc + b_col).astype(o_ref.dtype)


def kernel(x, weight, bias):
    B, K = x.shape
    N = weight.shape[0]

    xt = x.T

    tn = 32768
    while B % tn:
        tn //= 2
    grid = (B // tn,)

    outt = pl.pallas_call(
        _qhead_kernel,
        out_shape=jax.ShapeDtypeStruct((N, B), x.dtype),
        grid=grid,
        in_specs=[
            pl.BlockSpec((K, tn), lambda i: (0, i)),
            pl.BlockSpec((N, K), lambda i: (0, 0)),
            pl.BlockSpec((1, N), lambda i: (0, 0)),
        ],
        out_specs=pl.BlockSpec((N, tn), lambda i: (0, i)),
        compiler_params=pltpu.CompilerParams(
            dimension_semantics=("parallel",),
            vmem_limit_bytes=64 * 1024 * 1024,
        ),
    )(xt, weight, bias.reshape(1, N))

    return outt.T
